# baseline (speedup 1.0000x reference)
import functools

import jax
import jax.numpy as jnp
from jax import lax
from jax.experimental import pallas as pl
from jax.experimental.pallas import tpu as pltpu
from jax.experimental.pallas import tpu_sc as plsc

_CHUNK = 256


def _tc_retile(int2, n, cin):
    bt = 32
    nb = (n // 128 + bt - 1) // bt

    def body(g_ref, o_ref):
        for t in range(bt):
            o_ref[t] = g_ref[:, t * 128 : (t + 1) * 128]

    return pl.pallas_call(
        body,
        grid=(nb,),
        in_specs=[pl.BlockSpec((cin, bt * 128), lambda i: (0, i))],
        out_specs=pl.BlockSpec((bt, cin, 128), lambda i: (i, 0, 0)),
        out_shape=jax.ShapeDtypeStruct((n // 128, cin, 128), jnp.float32),
        compiler_params=pltpu.CompilerParams(
            dimension_semantics=("arbitrary",),
        ),
    )(int2)


def _sc_transpose(int4, n, cin):
    info = plsc.get_sparse_core_info()
    nc, ns = info.num_cores, info.num_subcores
    nw = nc * ns
    csz = 2048
    nchunk = (n + csz - 1) // csz
    per_w = (nchunk + nw - 1) // nw
    last_base = n - csz

    mesh = plsc.VectorSubcoreMesh(core_axis_name="c", subcore_axis_name="s")

    @functools.partial(
        pl.kernel,
        out_type=jax.ShapeDtypeStruct((n, cin), jnp.float32),
        mesh=mesh,
        scratch_types=[
            pltpu.VMEM((csz // 128, cin, 128), jnp.float32),
            pltpu.VMEM((csz, cin), jnp.float32),
        ],
        compiler_params=pltpu.CompilerParams(
            use_tc_tiling_on_sc=False, needs_layout_passes=False
        ),
    )
    def transpose_kernel(int_hbm, tab_hbm, tin, tout):
        wid = lax.axis_index("s") * nc + lax.axis_index("c")
        iota = lax.iota(jnp.int32, 16)
        cvecs = [jnp.full((16,), c, jnp.int32) for c in range(cin)]

        def body(q, carry):
            b = jnp.minimum((wid + q * nw) * csz, last_base)
            pltpu.sync_copy(
                int_hbm.at[pl.ds(b // 128, csz // 128), :, :], tin
            )

            def tbody(g, c2):
                pidx = g * 16 + iota
                t = g // 8
                lo = (g % 8) * 16
                for c in range(cin):
                    v = tin[t, c, pl.ds(lo, 16)]
                    plsc.store_scatter(tout, [pidx, cvecs[c]], v)
                return c2

            lax.fori_loop(0, csz // 16, tbody, 0)
            pltpu.sync_copy(tout, tab_hbm.at[pl.ds(b, csz), :])
            return carry

        lax.fori_loop(0, per_w, body, 0)

    return transpose_kernel(int4)


def _sc_gather(nbr_flat, table, k2, n, cin, start, count):
    info = plsc.get_sparse_core_info()
    nc, ns = info.num_cores, info.num_subcores
    nw = nc * ns
    chunk = _CHUNK
    nchunk = (count + chunk - 1) // chunk
    per_w = (nchunk + nw - 1) // nw
    jc = k2 * cin
    tpc = chunk // 128
    last_base = count - chunk
    ng = chunk // 16

    mesh = plsc.VectorSubcoreMesh(core_axis_name="c", subcore_axis_name="s")

    @functools.partial(
        pl.kernel,
        out_type=jax.ShapeDtypeStruct(
            (jc // 8, count // 128, 8, 128), jnp.float32
        ),
        mesh=mesh,
        scratch_types=[
            pltpu.VMEM((chunk * k2,), jnp.int32),
            pltpu.VMEM((chunk * k2,), jnp.int32),
            pltpu.VMEM((chunk * k2, cin), jnp.float32),
            pltpu.VMEM((chunk * k2, cin), jnp.float32),
            pltpu.VMEM((jc // 8, tpc, 8, 128), jnp.float32),
            pltpu.VMEM((jc // 8, tpc, 8, 128), jnp.float32),
            pltpu.SemaphoreType.DMA,
            pltpu.SemaphoreType.DMA,
            pltpu.SemaphoreType.DMA,
            pltpu.SemaphoreType.DMA,
        ],
        compiler_params=pltpu.CompilerParams(
            use_tc_tiling_on_sc=False, needs_layout_passes=False
        ),
    )
    def gather_kernel(
        nbr_hbm, tab_hbm, g_hbm,
        idx0, idx1, gb0, gb1, gtA, gtB, sem0, sem1, semsA, semsB,
    ):
        wid = lax.axis_index("s") * nc + lax.axis_index("c")
        iota9 = lax.iota(jnp.int32, 16) * k2
        cvecs = [jnp.full((16,), c, jnp.int32) for c in range(cin)]

        def cbase(q):
            return jnp.minimum((wid + q * nw) * chunk, last_base)

        def fire(q, idxv, gb, sem):
            b = cbase(q) + start
            pltpu.sync_copy(nbr_hbm.at[pl.ds(b * k2, chunk * k2)], idxv)
            pltpu.async_copy(tab_hbm.at[idxv], gb, sem)

        def wait_g(idxv, gb, sem):
            pltpu.make_async_copy(tab_hbm.at[idxv], gb, sem).wait()

        def wait_st(gt, sem):
            pltpu.make_async_copy(
                gt, g_hbm.at[:, pl.ds(0, tpc), :, :], sem
            ).wait()

        def transpose(gb, gt):
            def tbody(g, c2):
                pbase = g * 16
                ro = pbase * k2
                t = g // 8
                lo = (g % 8) * 16
                for k in range(k2):
                    ridx = iota9 + (ro + k)
                    for c in range(cin):
                        v = plsc.load_gather(gb, [ridx, cvecs[c]])
                        j = k * cin + c
                        gt[j // 8, t, j % 8, pl.ds(lo, 16)] = v
                return c2

            lax.fori_loop(0, ng, tbody, 0)

        def store(q, gt, sem):
            b = cbase(q)
            pltpu.async_copy(gt, g_hbm.at[:, pl.ds(b // 128, tpc), :, :], sem)

        fire(0, idx0, gb0, sem0)

        def body(p, carry):
            q0 = 2 * p
            fire(q0 + 1, idx1, gb1, sem1)
            wait_g(idx0, gb0, sem0)

            @pl.when(p > 0)
            def _():
                wait_st(gtA, semsA)

            transpose(gb0, gtA)
            store(q0, gtA, semsA)

            fire(q0 + 2, idx0, gb0, sem0)
            wait_g(idx1, gb1, sem1)

            @pl.when(p > 0)
            def _():
                wait_st(gtB, semsB)

            transpose(gb1, gtB)
            store(q0 + 1, gtB, semsB)
            return carry

        lax.fori_loop(0, (per_w - 1) // 2, body, 0)
        wait_g(idx0, gb0, sem0)
        wait_st(gtA, semsA)
        transpose(gb0, gtA)
        store(per_w - 1, gtA, semsA)
        wait_st(gtA, semsA)
        wait_st(gtB, semsB)

    assert per_w % 2 == 1
    return gather_kernel(nbr_flat, table)


def _tc_apply(g4, w32, ld2, bias2, n, jc, nstencils, cout):
    bt = 32
    bn = bt * 128
    nb = (n + bn - 1) // bn

    def body(g_ref, w_ref, ld_ref, b_ref, o_ref):
        for t in range(bt):
            g72 = g_ref[:, t].reshape(jc, 128)
            acc = jnp.dot(w_ref[...], g72, preferred_element_type=jnp.float32)
            ld = ld_ref[:, t * 128 : (t + 1) * 128]
            out = jnp.zeros((cout, 128), jnp.float32)
            for s in range(nstencils):
                out = out + jnp.where(
                    ld == s, acc[s * cout : (s + 1) * cout, :], 0.0
                )
            o_ref[:, t * 128 : (t + 1) * 128] = out + b_ref[...]

    return pl.pallas_call(
        body,
        grid=(nb,),
        in_specs=[
            pl.BlockSpec((jc // 8, bt, 8, 128), lambda i: (0, i, 0, 0)),
            pl.BlockSpec((nstencils * cout, jc), lambda i: (0, 0)),
            pl.BlockSpec((1, bn), lambda i: (0, i)),
            pl.BlockSpec((cout, 1), lambda i: (0, 0)),
        ],
        out_specs=pl.BlockSpec((cout, bn), lambda i: (0, i)),
        out_shape=jax.ShapeDtypeStruct((cout, n), jnp.float32),
        compiler_params=pltpu.CompilerParams(
            dimension_semantics=("arbitrary",),
        ),
    )(g4, w32, ld2, bias2)


def kernel(intensities, weight, bias, neighbors, level_deltas):
    b, cin, n = intensities.shape
    cout, _, nstencils, kh, kw = weight.shape
    k2 = kh * kw
    jc = k2 * cin

    table = _sc_transpose(_tc_retile(intensities.reshape(cin, n), n, cin), n, cin)
    nbr_flat = neighbors.astype(jnp.int32).reshape(-1)
    ld2 = level_deltas.astype(jnp.int32).reshape(1, n)
    w32 = jnp.transpose(weight, (2, 0, 3, 4, 1)).reshape(nstencils * cout, jc)
    bias2 = bias.reshape(cout, 1)

    half = n // 2
    outs = []
    for h in range(2):
        g4 = _sc_gather(nbr_flat, table, k2, n, cin, h * half, half)
        ld_h = lax.slice(ld2, (0, h * half), (1, (h + 1) * half))
        outs.append(
            _tc_apply(g4, w32, ld_h, bias2, half, jc, nstencils, cout)
        )
    return jnp.concatenate(outs, axis=1).reshape(b, cout, n)


# speedup vs baseline: 1.0654x; 1.0654x over previous
import functools

import jax
import jax.numpy as jnp
from jax import lax
from jax.experimental import pallas as pl
from jax.experimental.pallas import tpu as pltpu
from jax.experimental.pallas import tpu_sc as plsc

_CHUNK = 384


def _tc_retile(int2, n, cin):
    bt = 32
    nb = (n // 128 + bt - 1) // bt

    def body(g_ref, o_ref):
        for t in range(bt):
            o_ref[t] = g_ref[:, t * 128 : (t + 1) * 128]

    return pl.pallas_call(
        body,
        grid=(nb,),
        in_specs=[pl.BlockSpec((cin, bt * 128), lambda i: (0, i))],
        out_specs=pl.BlockSpec((bt, cin, 128), lambda i: (i, 0, 0)),
        out_shape=jax.ShapeDtypeStruct((n // 128, cin, 128), jnp.float32),
        compiler_params=pltpu.CompilerParams(
            dimension_semantics=("arbitrary",),
        ),
    )(int2)


def _sc_transpose(int4, n, cin):
    info = plsc.get_sparse_core_info()
    nc, ns = info.num_cores, info.num_subcores
    nw = nc * ns
    csz = 2048
    nchunk = (n + csz - 1) // csz
    per_w = (nchunk + nw - 1) // nw
    last_base = n - csz

    mesh = plsc.VectorSubcoreMesh(core_axis_name="c", subcore_axis_name="s")

    @functools.partial(
        pl.kernel,
        out_type=jax.ShapeDtypeStruct((n, cin), jnp.float32),
        mesh=mesh,
        scratch_types=[
            pltpu.VMEM((csz // 128, cin, 128), jnp.float32),
            pltpu.VMEM((csz, cin), jnp.float32),
        ],
        compiler_params=pltpu.CompilerParams(
            use_tc_tiling_on_sc=False, needs_layout_passes=False
        ),
    )
    def transpose_kernel(int_hbm, tab_hbm, tin, tout):
        wid = lax.axis_index("s") * nc + lax.axis_index("c")
        iota = lax.iota(jnp.int32, 16)
        cvecs = [jnp.full((16,), c, jnp.int32) for c in range(cin)]

        def body(q, carry):
            b = jnp.minimum((wid + q * nw) * csz, last_base)
            pltpu.sync_copy(
                int_hbm.at[pl.ds(b // 128, csz // 128), :, :], tin
            )

            def tbody(g, c2):
                pidx = g * 16 + iota
                t = g // 8
                lo = (g % 8) * 16
                for c in range(cin):
                    v = tin[t, c, pl.ds(lo, 16)]
                    plsc.store_scatter(tout, [pidx, cvecs[c]], v)
                return c2

            lax.fori_loop(0, csz // 16, tbody, 0)
            pltpu.sync_copy(tout, tab_hbm.at[pl.ds(b, csz), :])
            return carry

        lax.fori_loop(0, per_w, body, 0)

    return transpose_kernel(int4)


def _sc_gather(nbr_flat, table, k2, n, cin, start, count):
    info = plsc.get_sparse_core_info()
    nc, ns = info.num_cores, info.num_subcores
    nw = nc * ns
    chunk = _CHUNK
    nchunk = (count + chunk - 1) // chunk
    per_w = (nchunk + nw - 1) // nw
    jc = k2 * cin
    tpc = chunk // 128
    last_base = count - chunk
    ng = chunk // 16

    mesh = plsc.VectorSubcoreMesh(core_axis_name="c", subcore_axis_name="s")

    @functools.partial(
        pl.kernel,
        out_type=jax.ShapeDtypeStruct(
            (jc // 8, count // 128, 8, 128), jnp.float32
        ),
        mesh=mesh,
        scratch_types=[
            pltpu.VMEM((chunk * k2,), jnp.int32),
            pltpu.VMEM((chunk * k2,), jnp.int32),
            pltpu.VMEM((chunk * k2, cin), jnp.float32),
            pltpu.VMEM((chunk * k2, cin), jnp.float32),
            pltpu.VMEM((jc // 8, tpc, 8, 128), jnp.float32),
            pltpu.VMEM((jc // 8, tpc, 8, 128), jnp.float32),
            pltpu.SemaphoreType.DMA,
            pltpu.SemaphoreType.DMA,
            pltpu.SemaphoreType.DMA,
            pltpu.SemaphoreType.DMA,
            pltpu.SemaphoreType.DMA,
            pltpu.SemaphoreType.DMA,
        ],
        compiler_params=pltpu.CompilerParams(
            use_tc_tiling_on_sc=False, needs_layout_passes=False
        ),
    )
    def gather_kernel(
        nbr_hbm, tab_hbm, g_hbm,
        idx0, idx1, gb0, gb1, gtA, gtB,
        sem0, sem1, semsA, semsB, semi0, semi1,
    ):
        wid = lax.axis_index("s") * nc + lax.axis_index("c")
        iota9 = lax.iota(jnp.int32, 16) * k2
        cvecs = [jnp.full((16,), c, jnp.int32) for c in range(cin)]

        def cbase(q):
            return jnp.minimum((wid + q * nw) * chunk, last_base)

        def fire_idx(q, idxv, semi):
            b = cbase(q) + start
            pltpu.async_copy(
                nbr_hbm.at[pl.ds(b * k2, chunk * k2)], idxv, semi
            )

        def wait_i(idxv, semi):
            pltpu.make_async_copy(
                nbr_hbm.at[pl.ds(0, chunk * k2)], idxv, semi
            ).wait()

        def fire_gather(idxv, gb, sem):
            pltpu.async_copy(tab_hbm.at[idxv], gb, sem)

        def wait_g(idxv, gb, sem):
            pltpu.make_async_copy(tab_hbm.at[idxv], gb, sem).wait()

        def wait_st(gt, sem):
            pltpu.make_async_copy(
                gt, g_hbm.at[:, pl.ds(0, tpc), :, :], sem
            ).wait()

        def transpose(gb, gt):
            def tbody(g, c2):
                pbase = g * 16
                ro = pbase * k2
                t = g // 8
                lo = (g % 8) * 16
                for k in range(k2):
                    ridx = iota9 + (ro + k)
                    for c in range(cin):
                        v = plsc.load_gather(gb, [ridx, cvecs[c]])
                        j = k * cin + c
                        gt[j // 8, t, j % 8, pl.ds(lo, 16)] = v
                return c2

            lax.fori_loop(0, ng, tbody, 0)

        def store(q, gt, sem):
            b = cbase(q)
            pltpu.async_copy(gt, g_hbm.at[:, pl.ds(b // 128, tpc), :, :], sem)

        fire_idx(0, idx0, semi0)
        wait_i(idx0, semi0)
        fire_gather(idx0, gb0, sem0)
        fire_idx(1, idx1, semi1)

        def half_step(p, q, idxv, semi, gb, semg, gt, sems,
                      idxn, semin, gbn, semgn):
            wait_i(idxn, semin)
            fire_gather(idxn, gbn, semgn)
            wait_g(idxv, gb, semg)
            fire_idx(q + 2, idxv, semi)

            @pl.when(p > 0)
            def _():
                wait_st(gt, sems)

            transpose(gb, gt)
            store(q, gt, sems)

        def body(p, carry):
            q0 = 2 * p
            half_step(p, q0, idx0, semi0, gb0, sem0, gtA, semsA,
                      idx1, semi1, gb1, sem1)
            half_step(p, q0 + 1, idx1, semi1, gb1, sem1, gtB, semsB,
                      idx0, semi0, gb0, sem0)
            return carry

        lax.fori_loop(0, (per_w - 1) // 2, body, 0)
        wait_g(idx0, gb0, sem0)
        wait_st(gtA, semsA)
        transpose(gb0, gtA)
        store(per_w - 1, gtA, semsA)
        wait_i(idx1, semi1)
        wait_st(gtA, semsA)
        wait_st(gtB, semsB)

    assert per_w % 2 == 1
    return gather_kernel(nbr_flat, table)


def _tc_apply(g4, w32, ld2, bias2, n, jc, nstencils, cout):
    bt = 32
    bn = bt * 128
    nb = (n + bn - 1) // bn

    def body(g_ref, w_ref, ld_ref, b_ref, o_ref):
        for t in range(bt):
            g72 = g_ref[:, t].reshape(jc, 128)
            acc = jnp.dot(w_ref[...], g72, preferred_element_type=jnp.float32)
            ld = ld_ref[:, t * 128 : (t + 1) * 128]
            out = jnp.zeros((cout, 128), jnp.float32)
            for s in range(nstencils):
                out = out + jnp.where(
                    ld == s, acc[s * cout : (s + 1) * cout, :], 0.0
                )
            o_ref[:, t * 128 : (t + 1) * 128] = out + b_ref[...]

    return pl.pallas_call(
        body,
        grid=(nb,),
        in_specs=[
            pl.BlockSpec((jc // 8, bt, 8, 128), lambda i: (0, i, 0, 0)),
            pl.BlockSpec((nstencils * cout, jc), lambda i: (0, 0)),
            pl.BlockSpec((1, bn), lambda i: (0, i)),
            pl.BlockSpec((cout, 1), lambda i: (0, 0)),
        ],
        out_specs=pl.BlockSpec((cout, bn), lambda i: (0, i)),
        out_shape=jax.ShapeDtypeStruct((cout, n), jnp.float32),
        compiler_params=pltpu.CompilerParams(
            dimension_semantics=("arbitrary",),
        ),
    )(g4, w32, ld2, bias2)


def kernel(intensities, weight, bias, neighbors, level_deltas):
    b, cin, n = intensities.shape
    cout, _, nstencils, kh, kw = weight.shape
    k2 = kh * kw
    jc = k2 * cin

    table = _sc_transpose(_tc_retile(intensities.reshape(cin, n), n, cin), n, cin)
    nbr_flat = neighbors.astype(jnp.int32).reshape(-1)
    ld2 = level_deltas.astype(jnp.int32).reshape(1, n)
    w32 = jnp.transpose(weight, (2, 0, 3, 4, 1)).reshape(nstencils * cout, jc)
    bias2 = bias.reshape(cout, 1)

    half = n // 2
    outs = []
    for h in range(2):
        g4 = _sc_gather(nbr_flat, table, k2, n, cin, h * half, half)
        ld_h = lax.slice(ld2, (0, h * half), (1, (h + 1) * half))
        outs.append(
            _tc_apply(g4, w32, ld_h, bias2, half, jc, nstencils, cout)
        )
    return jnp.concatenate(outs, axis=1).reshape(b, cout, n)


# speedup vs baseline: 1.1081x; 1.0401x over previous
import functools

import jax
import jax.numpy as jnp
from jax import lax
from jax.experimental import pallas as pl
from jax.experimental.pallas import tpu as pltpu
from jax.experimental.pallas import tpu_sc as plsc

_CHUNK = 384


def _tc_retile(int2, n, cin):
    bt = 64
    nb = (n // 128 + bt - 1) // bt

    def body(g_ref, o_ref):
        for t in range(bt):
            o_ref[t] = g_ref[:, t * 128 : (t + 1) * 128]

    return pl.pallas_call(
        body,
        grid=(nb,),
        in_specs=[pl.BlockSpec((cin, bt * 128), lambda i: (0, i))],
        out_specs=pl.BlockSpec((bt, cin, 128), lambda i: (i, 0, 0)),
        out_shape=jax.ShapeDtypeStruct((n // 128, cin, 128), jnp.float32),
        compiler_params=pltpu.CompilerParams(
            dimension_semantics=("arbitrary",),
        ),
    )(int2)


def _sc_transpose(int4, n, cin):
    info = plsc.get_sparse_core_info()
    nc, ns = info.num_cores, info.num_subcores
    nw = nc * ns
    csz = 2048
    nchunk = (n + csz - 1) // csz
    per_w = (nchunk + nw - 1) // nw
    last_base = n - csz

    mesh = plsc.VectorSubcoreMesh(core_axis_name="c", subcore_axis_name="s")

    @functools.partial(
        pl.kernel,
        out_type=jax.ShapeDtypeStruct((n, cin), jnp.float32),
        mesh=mesh,
        scratch_types=[
            pltpu.VMEM((csz // 128, cin, 128), jnp.float32),
            pltpu.VMEM((csz, cin), jnp.float32),
        ],
        compiler_params=pltpu.CompilerParams(
            use_tc_tiling_on_sc=False, needs_layout_passes=False
        ),
    )
    def transpose_kernel(int_hbm, tab_hbm, tin, tout):
        wid = lax.axis_index("s") * nc + lax.axis_index("c")
        iota = lax.iota(jnp.int32, 16)
        cvecs = [jnp.full((16,), c, jnp.int32) for c in range(cin)]

        def body(q, carry):
            b = jnp.minimum((wid + q * nw) * csz, last_base)
            pltpu.sync_copy(
                int_hbm.at[pl.ds(b // 128, csz // 128), :, :], tin
            )

            def tbody(g, c2):
                pidx = g * 16 + iota
                t = g // 8
                lo = (g % 8) * 16
                for c in range(cin):
                    v = tin[t, c, pl.ds(lo, 16)]
                    plsc.store_scatter(tout, [pidx, cvecs[c]], v)
                return c2

            lax.fori_loop(0, csz // 16, tbody, 0)
            pltpu.sync_copy(tout, tab_hbm.at[pl.ds(b, csz), :])
            return carry

        lax.fori_loop(0, per_w, body, 0)

    return transpose_kernel(int4)


def _sc_gather(nbr_flat, table, k2, n, cin, start, count):
    info = plsc.get_sparse_core_info()
    nc, ns = info.num_cores, info.num_subcores
    nw = nc * ns
    chunk = _CHUNK
    nchunk = (count + chunk - 1) // chunk
    per_w = (nchunk + nw - 1) // nw
    jc = k2 * cin
    tpc = chunk // 128
    last_base = count - chunk
    ng = chunk // 16

    mesh = plsc.VectorSubcoreMesh(core_axis_name="c", subcore_axis_name="s")

    @functools.partial(
        pl.kernel,
        out_type=jax.ShapeDtypeStruct(
            (jc // 8, count // 128, 8, 128), jnp.float32
        ),
        mesh=mesh,
        scratch_types=[
            pltpu.VMEM((chunk * k2,), jnp.int32),
            pltpu.VMEM((chunk * k2,), jnp.int32),
            pltpu.VMEM((chunk * k2, cin), jnp.float32),
            pltpu.VMEM((chunk * k2, cin), jnp.float32),
            pltpu.VMEM((jc // 8, tpc, 8, 128), jnp.float32),
            pltpu.VMEM((jc // 8, tpc, 8, 128), jnp.float32),
            pltpu.SemaphoreType.DMA,
            pltpu.SemaphoreType.DMA,
            pltpu.SemaphoreType.DMA,
            pltpu.SemaphoreType.DMA,
            pltpu.SemaphoreType.DMA,
            pltpu.SemaphoreType.DMA,
        ],
        compiler_params=pltpu.CompilerParams(
            use_tc_tiling_on_sc=False, needs_layout_passes=False
        ),
    )
    def gather_kernel(
        nbr_hbm, tab_hbm, g_hbm,
        idx0, idx1, gb0, gb1, gtA, gtB,
        sem0, sem1, semsA, semsB, semi0, semi1,
    ):
        wid = lax.axis_index("s") * nc + lax.axis_index("c")
        iota9 = lax.iota(jnp.int32, 16) * k2
        cvecs = [jnp.full((16,), c, jnp.int32) for c in range(cin)]

        def cbase(q):
            return jnp.minimum((wid + q * nw) * chunk, last_base)

        def fire_idx(q, idxv, semi):
            b = cbase(q) + start
            pltpu.async_copy(
                nbr_hbm.at[pl.ds(b * k2, chunk * k2)], idxv, semi
            )

        def wait_i(idxv, semi):
            pltpu.make_async_copy(
                nbr_hbm.at[pl.ds(0, chunk * k2)], idxv, semi
            ).wait()

        def fire_gather(idxv, gb, sem):
            pltpu.async_copy(tab_hbm.at[idxv], gb, sem)

        def wait_g(idxv, gb, sem):
            pltpu.make_async_copy(tab_hbm.at[idxv], gb, sem).wait()

        def wait_st(gt, sem):
            pltpu.make_async_copy(
                gt, g_hbm.at[:, pl.ds(0, tpc), :, :], sem
            ).wait()

        def transpose(gb, gt):
            def tbody(g, c2):
                pbase = g * 16
                ro = pbase * k2
                t = g // 8
                lo = (g % 8) * 16
                for k in range(k2):
                    ridx = iota9 + (ro + k)
                    for c in range(cin):
                        v = plsc.load_gather(gb, [ridx, cvecs[c]])
                        j = k * cin + c
                        gt[j // 8, t, j % 8, pl.ds(lo, 16)] = v
                return c2

            lax.fori_loop(0, ng, tbody, 0)

        def store(q, gt, sem):
            b = cbase(q)
            pltpu.async_copy(gt, g_hbm.at[:, pl.ds(b // 128, tpc), :, :], sem)

        fire_idx(0, idx0, semi0)
        wait_i(idx0, semi0)
        fire_gather(idx0, gb0, sem0)
        fire_idx(1, idx1, semi1)

        def half_step(p, q, idxv, semi, gb, semg, gt, sems,
                      idxn, semin, gbn, semgn):
            wait_i(idxn, semin)
            fire_gather(idxn, gbn, semgn)
            wait_g(idxv, gb, semg)
            fire_idx(q + 2, idxv, semi)

            @pl.when(p > 0)
            def _():
                wait_st(gt, sems)

            transpose(gb, gt)
            store(q, gt, sems)

        def body(p, carry):
            q0 = 2 * p
            half_step(p, q0, idx0, semi0, gb0, sem0, gtA, semsA,
                      idx1, semi1, gb1, sem1)
            half_step(p, q0 + 1, idx1, semi1, gb1, sem1, gtB, semsB,
                      idx0, semi0, gb0, sem0)
            return carry

        lax.fori_loop(0, (per_w - 1) // 2, body, 0)
        wait_g(idx0, gb0, sem0)
        wait_st(gtA, semsA)
        transpose(gb0, gtA)
        store(per_w - 1, gtA, semsA)
        wait_i(idx1, semi1)
        wait_st(gtA, semsA)
        wait_st(gtB, semsB)

    assert per_w % 2 == 1
    return gather_kernel(nbr_flat, table)


def _tc_apply(g4, w32, ld2, bias2, n, jc, nstencils, cout):
    bt = 32
    bn = bt * 128
    nb = (n + bn - 1) // bn

    def body(g_ref, w_ref, ld_ref, b_ref, o_ref):
        for t in range(bt):
            g72 = g_ref[:, t].reshape(jc, 128)
            acc = jnp.dot(w_ref[...], g72, preferred_element_type=jnp.float32)
            ld = ld_ref[:, t * 128 : (t + 1) * 128]
            out = jnp.zeros((cout, 128), jnp.float32)
            for s in range(nstencils):
                out = out + jnp.where(
                    ld == s, acc[s * cout : (s + 1) * cout, :], 0.0
                )
            o_ref[:, t * 128 : (t + 1) * 128] = out + b_ref[...]

    return pl.pallas_call(
        body,
        grid=(nb,),
        in_specs=[
            pl.BlockSpec((jc // 8, bt, 8, 128), lambda i: (0, i, 0, 0)),
            pl.BlockSpec((nstencils * cout, jc), lambda i: (0, 0)),
            pl.BlockSpec((1, bn), lambda i: (0, i)),
            pl.BlockSpec((cout, 1), lambda i: (0, 0)),
        ],
        out_specs=pl.BlockSpec((cout, bn), lambda i: (0, i)),
        out_shape=jax.ShapeDtypeStruct((cout, n), jnp.float32),
        compiler_params=pltpu.CompilerParams(
            dimension_semantics=("arbitrary",),
        ),
    )(g4, w32, ld2, bias2)


def kernel(intensities, weight, bias, neighbors, level_deltas):
    b, cin, n = intensities.shape
    cout, _, nstencils, kh, kw = weight.shape
    k2 = kh * kw
    jc = k2 * cin

    table = _sc_transpose(_tc_retile(intensities.reshape(cin, n), n, cin), n, cin)
    nbr_flat = neighbors.astype(jnp.int32).reshape(-1)
    ld2 = level_deltas.astype(jnp.int32).reshape(1, n)
    w32 = jnp.transpose(weight, (2, 0, 3, 4, 1)).reshape(nstencils * cout, jc)
    bias2 = bias.reshape(cout, 1)

    half = n // 2
    outs = []
    for h in range(2):
        g4 = _sc_gather(nbr_flat, table, k2, n, cin, h * half, half)
        ld_h = lax.slice(ld2, (0, h * half), (1, (h + 1) * half))
        outs.append(
            _tc_apply(g4, w32, ld_h, bias2, half, jc, nstencils, cout)
        )
    return jnp.concatenate(outs, axis=1).reshape(b, cout, n)


# speedup vs baseline: 1.1360x; 1.0251x over previous
import functools

import jax
import jax.numpy as jnp
from jax import lax
from jax.experimental import pallas as pl
from jax.experimental.pallas import tpu as pltpu
from jax.experimental.pallas import tpu_sc as plsc

_CHUNK = 384


def _tc_retile(int2, n, cin):
    bt = 128
    nb = (n // 128 + bt - 1) // bt

    def body(g_ref, o_ref):
        for t in range(bt):
            o_ref[t] = g_ref[:, t * 128 : (t + 1) * 128]

    return pl.pallas_call(
        body,
        grid=(nb,),
        in_specs=[pl.BlockSpec((cin, bt * 128), lambda i: (0, i))],
        out_specs=pl.BlockSpec((bt, cin, 128), lambda i: (i, 0, 0)),
        out_shape=jax.ShapeDtypeStruct((n // 128, cin, 128), jnp.float32),
        compiler_params=pltpu.CompilerParams(
            dimension_semantics=("arbitrary",),
        ),
    )(int2)


def _sc_transpose(int4, n, cin):
    info = plsc.get_sparse_core_info()
    nc, ns = info.num_cores, info.num_subcores
    nw = nc * ns
    csz = 2048
    nchunk = (n + csz - 1) // csz
    per_w = (nchunk + nw - 1) // nw
    last_base = n - csz

    mesh = plsc.VectorSubcoreMesh(core_axis_name="c", subcore_axis_name="s")

    @functools.partial(
        pl.kernel,
        out_type=jax.ShapeDtypeStruct((n, cin), jnp.float32),
        mesh=mesh,
        scratch_types=[
            pltpu.VMEM((csz // 128, cin, 128), jnp.float32),
            pltpu.VMEM((csz, cin), jnp.float32),
        ],
        compiler_params=pltpu.CompilerParams(
            use_tc_tiling_on_sc=False, needs_layout_passes=False
        ),
    )
    def transpose_kernel(int_hbm, tab_hbm, tin, tout):
        wid = lax.axis_index("s") * nc + lax.axis_index("c")
        iota = lax.iota(jnp.int32, 16)
        cvecs = [jnp.full((16,), c, jnp.int32) for c in range(cin)]

        def body(q, carry):
            b = jnp.minimum((wid + q * nw) * csz, last_base)
            pltpu.sync_copy(
                int_hbm.at[pl.ds(b // 128, csz // 128), :, :], tin
            )

            def tbody(g, c2):
                pidx = g * 16 + iota
                t = g // 8
                lo = (g % 8) * 16
                for c in range(cin):
                    v = tin[t, c, pl.ds(lo, 16)]
                    plsc.store_scatter(tout, [pidx, cvecs[c]], v)
                return c2

            lax.fori_loop(0, csz // 16, tbody, 0)
            pltpu.sync_copy(tout, tab_hbm.at[pl.ds(b, csz), :])
            return carry

        lax.fori_loop(0, per_w, body, 0)

    return transpose_kernel(int4)


def _sc_gather(nbr_flat, table, k2, n, cin, start, count):
    info = plsc.get_sparse_core_info()
    nc, ns = info.num_cores, info.num_subcores
    nw = nc * ns
    chunk = _CHUNK
    nchunk = (count + chunk - 1) // chunk
    per_w = (nchunk + nw - 1) // nw
    jc = k2 * cin
    tpc = chunk // 128
    last_base = count - chunk
    ng = chunk // 16

    mesh = plsc.VectorSubcoreMesh(core_axis_name="c", subcore_axis_name="s")

    @functools.partial(
        pl.kernel,
        out_type=jax.ShapeDtypeStruct(
            (jc // 8, count // 128, 8, 128), jnp.float32
        ),
        mesh=mesh,
        scratch_types=[
            pltpu.VMEM((chunk * k2,), jnp.int32),
            pltpu.VMEM((chunk * k2,), jnp.int32),
            pltpu.VMEM((chunk * k2, cin), jnp.float32),
            pltpu.VMEM((chunk * k2, cin), jnp.float32),
            pltpu.VMEM((jc // 8, tpc, 8, 128), jnp.float32),
            pltpu.VMEM((jc // 8, tpc, 8, 128), jnp.float32),
            pltpu.SemaphoreType.DMA,
            pltpu.SemaphoreType.DMA,
            pltpu.SemaphoreType.DMA,
            pltpu.SemaphoreType.DMA,
            pltpu.SemaphoreType.DMA,
            pltpu.SemaphoreType.DMA,
        ],
        compiler_params=pltpu.CompilerParams(
            use_tc_tiling_on_sc=False, needs_layout_passes=False
        ),
    )
    def gather_kernel(
        nbr_hbm, tab_hbm, g_hbm,
        idx0, idx1, gb0, gb1, gtA, gtB,
        sem0, sem1, semsA, semsB, semi0, semi1,
    ):
        wid = lax.axis_index("s") * nc + lax.axis_index("c")
        iota9 = lax.iota(jnp.int32, 16) * k2
        cvecs = [jnp.full((16,), c, jnp.int32) for c in range(cin)]

        def cbase(q):
            return jnp.minimum((wid + q * nw) * chunk, last_base)

        def fire_idx(q, idxv, semi):
            b = cbase(q) + start
            pltpu.async_copy(
                nbr_hbm.at[pl.ds(b * k2, chunk * k2)], idxv, semi
            )

        def wait_i(idxv, semi):
            pltpu.make_async_copy(
                nbr_hbm.at[pl.ds(0, chunk * k2)], idxv, semi
            ).wait()

        def fire_gather(idxv, gb, sem):
            pltpu.async_copy(tab_hbm.at[idxv], gb, sem)

        def wait_g(idxv, gb, sem):
            pltpu.make_async_copy(tab_hbm.at[idxv], gb, sem).wait()

        def wait_st(gt, sem):
            pltpu.make_async_copy(
                gt, g_hbm.at[:, pl.ds(0, tpc), :, :], sem
            ).wait()

        def transpose(gb, gt):
            def tbody(g, c2):
                pbase = g * 16
                ro = pbase * k2
                t = g // 8
                lo = (g % 8) * 16
                for k in range(k2):
                    ridx = iota9 + (ro + k)
                    for c in range(cin):
                        v = plsc.load_gather(gb, [ridx, cvecs[c]])
                        j = k * cin + c
                        gt[j // 8, t, j % 8, pl.ds(lo, 16)] = v
                return c2

            lax.fori_loop(0, ng, tbody, 0)

        def store(q, gt, sem):
            b = cbase(q)
            pltpu.async_copy(gt, g_hbm.at[:, pl.ds(b // 128, tpc), :, :], sem)

        fire_idx(0, idx0, semi0)
        wait_i(idx0, semi0)
        fire_gather(idx0, gb0, sem0)
        fire_idx(1, idx1, semi1)

        def half_step(p, q, idxv, semi, gb, semg, gt, sems,
                      idxn, semin, gbn, semgn):
            wait_i(idxn, semin)
            fire_gather(idxn, gbn, semgn)
            wait_g(idxv, gb, semg)
            fire_idx(q + 2, idxv, semi)

            @pl.when(p > 0)
            def _():
                wait_st(gt, sems)

            transpose(gb, gt)
            store(q, gt, sems)

        def body(p, carry):
            q0 = 2 * p
            half_step(p, q0, idx0, semi0, gb0, sem0, gtA, semsA,
                      idx1, semi1, gb1, sem1)
            half_step(p, q0 + 1, idx1, semi1, gb1, sem1, gtB, semsB,
                      idx0, semi0, gb0, sem0)
            return carry

        lax.fori_loop(0, (per_w - 1) // 2, body, 0)
        wait_g(idx0, gb0, sem0)
        wait_st(gtA, semsA)
        transpose(gb0, gtA)
        store(per_w - 1, gtA, semsA)
        wait_i(idx1, semi1)
        wait_st(gtA, semsA)
        wait_st(gtB, semsB)

    assert per_w % 2 == 1
    return gather_kernel(nbr_flat, table)


def _tc_apply(g4, w32, ld2, bias2, n, jc, nstencils, cout):
    bt = 64
    bn = bt * 128
    nb = (n + bn - 1) // bn

    def body(g_ref, w_ref, ld_ref, b_ref, o_ref):
        for t in range(bt):
            g72 = g_ref[:, t].reshape(jc, 128)
            acc = jnp.dot(w_ref[...], g72, preferred_element_type=jnp.float32)
            ld = ld_ref[:, t * 128 : (t + 1) * 128]
            out = jnp.zeros((cout, 128), jnp.float32)
            for s in range(nstencils):
                out = out + jnp.where(
                    ld == s, acc[s * cout : (s + 1) * cout, :], 0.0
                )
            o_ref[:, t * 128 : (t + 1) * 128] = out + b_ref[...]

    return pl.pallas_call(
        body,
        grid=(nb,),
        in_specs=[
            pl.BlockSpec((jc // 8, bt, 8, 128), lambda i: (0, i, 0, 0)),
            pl.BlockSpec((nstencils * cout, jc), lambda i: (0, 0)),
            pl.BlockSpec((1, bn), lambda i: (0, i)),
            pl.BlockSpec((cout, 1), lambda i: (0, 0)),
        ],
        out_specs=pl.BlockSpec((cout, bn), lambda i: (0, i)),
        out_shape=jax.ShapeDtypeStruct((cout, n), jnp.float32),
        compiler_params=pltpu.CompilerParams(
            dimension_semantics=("arbitrary",),
        ),
    )(g4, w32, ld2, bias2)


def kernel(intensities, weight, bias, neighbors, level_deltas):
    b, cin, n = intensities.shape
    cout, _, nstencils, kh, kw = weight.shape
    k2 = kh * kw
    jc = k2 * cin

    table = _sc_transpose(_tc_retile(intensities.reshape(cin, n), n, cin), n, cin)
    nbr_flat = neighbors.astype(jnp.int32).reshape(-1)
    ld2 = level_deltas.astype(jnp.int32).reshape(1, n)
    w32 = jnp.transpose(weight, (2, 0, 3, 4, 1)).reshape(nstencils * cout, jc)
    bias2 = bias.reshape(cout, 1)

    half = n // 2
    outs = []
    for h in range(2):
        g4 = _sc_gather(nbr_flat, table, k2, n, cin, h * half, half)
        ld_h = lax.slice(ld2, (0, h * half), (1, (h + 1) * half))
        outs.append(
            _tc_apply(g4, w32, ld_h, bias2, half, jc, nstencils, cout)
        )
    return jnp.concatenate(outs, axis=1).reshape(b, cout, n)


# speedup vs baseline: 1.1459x; 1.0087x over previous
import functools

import jax
import jax.numpy as jnp
from jax import lax
from jax.experimental import pallas as pl
from jax.experimental.pallas import tpu as pltpu
from jax.experimental.pallas import tpu_sc as plsc

_CHUNK = 384


def _tc_retile(int2, n, cin):
    bt = 256
    nb = (n // 128 + bt - 1) // bt

    def body(g_ref, o_ref):
        for t in range(bt):
            o_ref[t] = g_ref[:, t * 128 : (t + 1) * 128]

    return pl.pallas_call(
        body,
        grid=(nb,),
        in_specs=[pl.BlockSpec((cin, bt * 128), lambda i: (0, i))],
        out_specs=pl.BlockSpec((bt, cin, 128), lambda i: (i, 0, 0)),
        out_shape=jax.ShapeDtypeStruct((n // 128, cin, 128), jnp.float32),
        compiler_params=pltpu.CompilerParams(
            dimension_semantics=("arbitrary",),
        ),
    )(int2)


def _sc_transpose(int4, n, cin):
    info = plsc.get_sparse_core_info()
    nc, ns = info.num_cores, info.num_subcores
    nw = nc * ns
    csz = 2048
    nchunk = (n + csz - 1) // csz
    per_w = (nchunk + nw - 1) // nw
    last_base = n - csz

    mesh = plsc.VectorSubcoreMesh(core_axis_name="c", subcore_axis_name="s")

    @functools.partial(
        pl.kernel,
        out_type=jax.ShapeDtypeStruct((n, cin), jnp.float32),
        mesh=mesh,
        scratch_types=[
            pltpu.VMEM((csz // 128, cin, 128), jnp.float32),
            pltpu.VMEM((csz, cin), jnp.float32),
        ],
        compiler_params=pltpu.CompilerParams(
            use_tc_tiling_on_sc=False, needs_layout_passes=False
        ),
    )
    def transpose_kernel(int_hbm, tab_hbm, tin, tout):
        wid = lax.axis_index("s") * nc + lax.axis_index("c")
        iota = lax.iota(jnp.int32, 16)
        cvecs = [jnp.full((16,), c, jnp.int32) for c in range(cin)]

        def body(q, carry):
            b = jnp.minimum((wid + q * nw) * csz, last_base)
            pltpu.sync_copy(
                int_hbm.at[pl.ds(b // 128, csz // 128), :, :], tin
            )

            def tbody(g, c2):
                pidx = g * 16 + iota
                t = g // 8
                lo = (g % 8) * 16
                for c in range(cin):
                    v = tin[t, c, pl.ds(lo, 16)]
                    plsc.store_scatter(tout, [pidx, cvecs[c]], v)
                return c2

            lax.fori_loop(0, csz // 16, tbody, 0)
            pltpu.sync_copy(tout, tab_hbm.at[pl.ds(b, csz), :])
            return carry

        lax.fori_loop(0, per_w, body, 0)

    return transpose_kernel(int4)


def _sc_gather(nbr_flat, table, k2, n, cin, start, count):
    info = plsc.get_sparse_core_info()
    nc, ns = info.num_cores, info.num_subcores
    nw = nc * ns
    chunk = _CHUNK
    nchunk = (count + chunk - 1) // chunk
    per_w = (nchunk + nw - 1) // nw
    jc = k2 * cin
    tpc = chunk // 128
    last_base = count - chunk
    ng = chunk // 16

    mesh = plsc.VectorSubcoreMesh(core_axis_name="c", subcore_axis_name="s")

    @functools.partial(
        pl.kernel,
        out_type=jax.ShapeDtypeStruct(
            (jc // 8, count // 128, 8, 128), jnp.float32
        ),
        mesh=mesh,
        scratch_types=[
            pltpu.VMEM((chunk * k2,), jnp.int32),
            pltpu.VMEM((chunk * k2,), jnp.int32),
            pltpu.VMEM((chunk * k2, cin), jnp.float32),
            pltpu.VMEM((chunk * k2, cin), jnp.float32),
            pltpu.VMEM((jc // 8, tpc, 8, 128), jnp.float32),
            pltpu.VMEM((jc // 8, tpc, 8, 128), jnp.float32),
            pltpu.SemaphoreType.DMA,
            pltpu.SemaphoreType.DMA,
            pltpu.SemaphoreType.DMA,
            pltpu.SemaphoreType.DMA,
            pltpu.SemaphoreType.DMA,
            pltpu.SemaphoreType.DMA,
        ],
        compiler_params=pltpu.CompilerParams(
            use_tc_tiling_on_sc=False, needs_layout_passes=False
        ),
    )
    def gather_kernel(
        nbr_hbm, tab_hbm, g_hbm,
        idx0, idx1, gb0, gb1, gtA, gtB,
        sem0, sem1, semsA, semsB, semi0, semi1,
    ):
        wid = lax.axis_index("s") * nc + lax.axis_index("c")
        iota9 = lax.iota(jnp.int32, 16) * k2
        cvecs = [jnp.full((16,), c, jnp.int32) for c in range(cin)]

        def cbase(q):
            return jnp.minimum((wid + q * nw) * chunk, last_base)

        def fire_idx(q, idxv, semi):
            b = cbase(q) + start
            pltpu.async_copy(
                nbr_hbm.at[pl.ds(b * k2, chunk * k2)], idxv, semi
            )

        def wait_i(idxv, semi):
            pltpu.make_async_copy(
                nbr_hbm.at[pl.ds(0, chunk * k2)], idxv, semi
            ).wait()

        def fire_gather(idxv, gb, sem):
            pltpu.async_copy(tab_hbm.at[idxv], gb, sem)

        def wait_g(idxv, gb, sem):
            pltpu.make_async_copy(tab_hbm.at[idxv], gb, sem).wait()

        def wait_st(gt, sem):
            pltpu.make_async_copy(
                gt, g_hbm.at[:, pl.ds(0, tpc), :, :], sem
            ).wait()

        def transpose(gb, gt):
            def tbody(g, c2):
                pbase = g * 16
                ro = pbase * k2
                t = g // 8
                lo = (g % 8) * 16
                for k in range(k2):
                    ridx = iota9 + (ro + k)
                    for c in range(cin):
                        v = plsc.load_gather(gb, [ridx, cvecs[c]])
                        j = k * cin + c
                        gt[j // 8, t, j % 8, pl.ds(lo, 16)] = v
                return c2

            lax.fori_loop(0, ng, tbody, 0)

        def store(q, gt, sem):
            b = cbase(q)
            pltpu.async_copy(gt, g_hbm.at[:, pl.ds(b // 128, tpc), :, :], sem)

        fire_idx(0, idx0, semi0)
        wait_i(idx0, semi0)
        fire_gather(idx0, gb0, sem0)
        fire_idx(1, idx1, semi1)

        def half_step(p, q, idxv, semi, gb, semg, gt, sems,
                      idxn, semin, gbn, semgn):
            wait_i(idxn, semin)
            fire_gather(idxn, gbn, semgn)
            wait_g(idxv, gb, semg)
            fire_idx(q + 2, idxv, semi)

            @pl.when(p > 0)
            def _():
                wait_st(gt, sems)

            transpose(gb, gt)
            store(q, gt, sems)

        def body(p, carry):
            q0 = 2 * p
            half_step(p, q0, idx0, semi0, gb0, sem0, gtA, semsA,
                      idx1, semi1, gb1, sem1)
            half_step(p, q0 + 1, idx1, semi1, gb1, sem1, gtB, semsB,
                      idx0, semi0, gb0, sem0)
            return carry

        lax.fori_loop(0, (per_w - 1) // 2, body, 0)
        wait_g(idx0, gb0, sem0)
        wait_st(gtA, semsA)
        transpose(gb0, gtA)
        store(per_w - 1, gtA, semsA)
        wait_i(idx1, semi1)
        wait_st(gtA, semsA)
        wait_st(gtB, semsB)

    assert per_w % 2 == 1
    return gather_kernel(nbr_flat, table)


def _tc_apply(g4, w32, ld2, bias2, n, jc, nstencils, cout):
    bt = 128
    bn = bt * 128
    nb = (n + bn - 1) // bn

    def body(g_ref, w_ref, ld_ref, b_ref, o_ref):
        for t in range(bt):
            g72 = g_ref[:, t].reshape(jc, 128)
            acc = jnp.dot(w_ref[...], g72, preferred_element_type=jnp.float32)
            ld = ld_ref[:, t * 128 : (t + 1) * 128]
            out = jnp.zeros((cout, 128), jnp.float32)
            for s in range(nstencils):
                out = out + jnp.where(
                    ld == s, acc[s * cout : (s + 1) * cout, :], 0.0
                )
            o_ref[:, t * 128 : (t + 1) * 128] = out + b_ref[...]

    return pl.pallas_call(
        body,
        grid=(nb,),
        in_specs=[
            pl.BlockSpec((jc // 8, bt, 8, 128), lambda i: (0, i, 0, 0)),
            pl.BlockSpec((nstencils * cout, jc), lambda i: (0, 0)),
            pl.BlockSpec((1, bn), lambda i: (0, i)),
            pl.BlockSpec((cout, 1), lambda i: (0, 0)),
        ],
        out_specs=pl.BlockSpec((cout, bn), lambda i: (0, i)),
        out_shape=jax.ShapeDtypeStruct((cout, n), jnp.float32),
        compiler_params=pltpu.CompilerParams(
            dimension_semantics=("arbitrary",),
        ),
    )(g4, w32, ld2, bias2)


def kernel(intensities, weight, bias, neighbors, level_deltas):
    b, cin, n = intensities.shape
    cout, _, nstencils, kh, kw = weight.shape
    k2 = kh * kw
    jc = k2 * cin

    table = _sc_transpose(_tc_retile(intensities.reshape(cin, n), n, cin), n, cin)
    nbr_flat = neighbors.astype(jnp.int32).reshape(-1)
    ld2 = level_deltas.astype(jnp.int32).reshape(1, n)
    w32 = jnp.transpose(weight, (2, 0, 3, 4, 1)).reshape(nstencils * cout, jc)
    bias2 = bias.reshape(cout, 1)

    half = n // 2
    outs = []
    for h in range(2):
        g4 = _sc_gather(nbr_flat, table, k2, n, cin, h * half, half)
        ld_h = lax.slice(ld2, (0, h * half), (1, (h + 1) * half))
        outs.append(
            _tc_apply(g4, w32, ld_h, bias2, half, jc, nstencils, cout)
        )
    return jnp.concatenate(outs, axis=1).reshape(b, cout, n)
